# baseline (speedup 1.0000x reference)
import functools

import jax
import jax.numpy as jnp
from jax import lax
from jax.experimental import pallas as pl
from jax.experimental.pallas import tpu as pltpu
from jax.experimental.pallas import tpu_sc as plsc

NC = 2
NS = 16
NW = NC * NS
LANES = 16
HALF = 250000


def _gather_body(bpw, L, tlo_hbm, thi_hbm, wordsT_hbm, out_hbm,
                 idx_v, idx_lo, idx_hi, gl_a, gl_b, gh_a, gh_b, ob_a, ob_b,
                 sg_a, sg_b, sw_a, sw_b):
    wid = lax.axis_index("s") * NC + lax.axis_index("c")
    b0 = pl.multiple_of(wid * bpw, bpw)
    n_pairs = bpw // 2
    n_idx = bpw * L

    pltpu.sync_copy(wordsT_hbm.at[:, pl.ds(b0, bpw)], idx_v)

    iota = lax.iota(jnp.int32, LANES)

    @plsc.parallel_loop(0, n_idx // LANES, step=1, unroll=8)
    def _(g):
        k = g * LANES + iota
        b = k // L
        l = k - b * L
        iv = plsc.load_gather(idx_v, [l, b])
        i2 = lax.shift_right_logical(iv, 1)
        plsc.store_scatter(idx_lo, [b, l], jnp.minimum(i2, HALF - 1))
        plsc.store_scatter(idx_hi, [b, l],
                           jnp.maximum(i2 - HALF, 0))

    def start_gather(b, gl, gh, sg):
        pltpu.async_copy(tlo_hbm.at[idx_lo.at[b]], gl, sg)
        pltpu.async_copy(thi_hbm.at[idx_hi.at[b]], gh, sg)

    def wait_gather(b, gl, gh, sg):
        pltpu.make_async_copy(tlo_hbm.at[idx_lo.at[b]], gl, sg).wait()
        pltpu.make_async_copy(thi_hbm.at[idx_hi.at[b]], gh, sg).wait()

    def extract(b, gl, gh, ob):
        bb = jnp.full((LANES,), 0, jnp.int32) + b

        @plsc.parallel_loop(0, L, step=1, unroll=8)
        def _(r):
            rr = jnp.full((LANES,), 0, jnp.int32) + r
            iv = plsc.load_gather(idx_v, [rr, bb])
            podd = (iv & 1) > 0
            phi = lax.shift_right_logical(iv, 1) >= HALF
            for c in range(4):
                l0 = gl[r, pl.ds(c * LANES, LANES)]
                l1 = gl[r, pl.ds(64 + c * LANES, LANES)]
                h0 = gh[r, pl.ds(c * LANES, LANES)]
                h1 = gh[r, pl.ds(64 + c * LANES, LANES)]
                ob[r, pl.ds(c * LANES, LANES)] = jnp.where(
                    phi, jnp.where(podd, h1, h0), jnp.where(podd, l1, l0))

    def start_wb(b, ob, sw):
        pltpu.async_copy(ob, out_hbm.at[b0 + b], sw)

    def wait_wb(ob, sw):
        pltpu.make_async_copy(ob, out_hbm.at[b0], sw).wait()

    start_gather(0, gl_a, gh_a, sg_a)

    def pair(p, carry):
        e = p * 2
        o = e + 1

        start_gather(o, gl_b, gh_b, sg_b)
        wait_gather(e, gl_a, gh_a, sg_a)

        @pl.when(p >= 1)
        def _():
            wait_wb(ob_a, sw_a)
        extract(e, gl_a, gh_a, ob_a)
        start_wb(e, ob_a, sw_a)

        @pl.when(p + 1 < n_pairs)
        def _():
            start_gather(e + 2, gl_a, gh_a, sg_a)

        wait_gather(o, gl_b, gh_b, sg_b)

        @pl.when(p >= 1)
        def _():
            wait_wb(ob_b, sw_b)
        extract(o, gl_b, gh_b, ob_b)
        start_wb(o, ob_b, sw_b)
        return carry

    lax.fori_loop(0, n_pairs, pair, 0)
    wait_wb(ob_a, sw_a)
    wait_wb(ob_b, sw_b)


@jax.jit
def kernel(words, word_emb):
    B, L = words.shape
    V, D = word_emb.shape
    if words.dtype != jnp.int32:
        words = words.astype(jnp.int32)

    table_lo = word_emb[:V // 2].reshape(HALF, 2 * D)
    table_hi = word_emb[V // 2:].reshape(HALF, 2 * D)
    wordsT = words.T

    mesh = plsc.VectorSubcoreMesh(core_axis_name="c", subcore_axis_name="s")
    bpw = B // NW
    body = functools.partial(_gather_body, bpw, L)
    out = pl.kernel(
        body,
        out_type=jax.ShapeDtypeStruct((B, L, D), jnp.float32),
        mesh=mesh,
        compiler_params=pltpu.CompilerParams(needs_layout_passes=False),
        scratch_types=[
            pltpu.VMEM((L, bpw), jnp.int32),
            pltpu.VMEM((bpw, L), jnp.int32),
            pltpu.VMEM((bpw, L), jnp.int32),
            pltpu.VMEM((L, 2 * D), jnp.float32),
            pltpu.VMEM((L, 2 * D), jnp.float32),
            pltpu.VMEM((L, 2 * D), jnp.float32),
            pltpu.VMEM((L, 2 * D), jnp.float32),
            pltpu.VMEM((L, D), jnp.float32),
            pltpu.VMEM((L, D), jnp.float32),
            pltpu.SemaphoreType.DMA,
            pltpu.SemaphoreType.DMA,
            pltpu.SemaphoreType.DMA,
            pltpu.SemaphoreType.DMA,
        ],
    )(table_lo, table_hi, wordsT)
    return out


# speedup vs baseline: 6.4144x; 6.4144x over previous
import functools

import jax
import jax.numpy as jnp
from jax import lax
from jax.experimental import pallas as pl
from jax.experimental.pallas import tpu as pltpu
from jax.experimental.pallas import tpu_sc as plsc

NC = 2
NS = 16
NW = NC * NS
LANES = 16


def _gather_body(bpw, L, table2_hbm, wordsT_hbm, out_hbm,
                 idx_v, idxT, gb_a, gb_b, ob_a, ob_b,
                 sg_a, sg_b, sw_a, sw_b):
    wid = lax.axis_index("s") * NC + lax.axis_index("c")
    b0 = pl.multiple_of(wid * bpw, bpw)
    n_pairs = bpw // 2
    n_idx = bpw * L

    pltpu.sync_copy(wordsT_hbm.at[:, pl.ds(b0, bpw)], idx_v)

    iota = lax.iota(jnp.int32, LANES)

    @plsc.parallel_loop(0, n_idx // LANES, step=1, unroll=8)
    def _(g):
        k = g * LANES + iota
        b = k // L
        l = k - b * L
        iv = plsc.load_gather(idx_v, [l, b])
        plsc.store_scatter(idxT, [b, l], lax.shift_right_logical(iv, 1))

    def start_gather(b, gb, sg):
        pltpu.async_copy(table2_hbm.at[idxT.at[b]], gb, sg)

    def wait_gather(b, gb, sg):
        pltpu.make_async_copy(table2_hbm.at[idxT.at[b]], gb, sg).wait()

    def extract(b, gb, ob):
        bb = jnp.full((LANES,), 0, jnp.int32) + b

        @plsc.parallel_loop(0, L, step=1, unroll=8)
        def _(r):
            rr = jnp.full((LANES,), 0, jnp.int32) + r
            hv = plsc.load_gather(idx_v, [rr, bb])
            pred = (hv & 1) > 0
            for c in range(4):
                v0 = gb[r, pl.ds(c * LANES, LANES)]
                v1 = gb[r, pl.ds(64 + c * LANES, LANES)]
                ob[r, pl.ds(c * LANES, LANES)] = jnp.where(pred, v1, v0)

    def start_wb(b, ob, sw):
        pltpu.async_copy(ob, out_hbm.at[b0 + b], sw)

    def wait_wb(ob, sw):
        pltpu.make_async_copy(ob, out_hbm.at[b0], sw).wait()

    start_gather(0, gb_a, sg_a)

    def pair(p, carry):
        e = p * 2
        o = e + 1

        start_gather(o, gb_b, sg_b)
        wait_gather(e, gb_a, sg_a)

        @pl.when(p >= 1)
        def _():
            wait_wb(ob_a, sw_a)
        extract(e, gb_a, ob_a)
        start_wb(e, ob_a, sw_a)

        @pl.when(p + 1 < n_pairs)
        def _():
            start_gather(e + 2, gb_a, sg_a)

        wait_gather(o, gb_b, sg_b)

        @pl.when(p >= 1)
        def _():
            wait_wb(ob_b, sw_b)
        extract(o, gb_b, ob_b)
        start_wb(o, ob_b, sw_b)
        return carry

    lax.fori_loop(0, n_pairs, pair, 0)
    wait_wb(ob_a, sw_a)
    wait_wb(ob_b, sw_b)


@jax.jit
def kernel(words, word_emb):
    B, L = words.shape
    V, D = word_emb.shape
    if words.dtype != jnp.int32:
        words = words.astype(jnp.int32)

    table2 = word_emb.reshape(V // 2, 2 * D)
    wordsT = words.T

    mesh = plsc.VectorSubcoreMesh(core_axis_name="c", subcore_axis_name="s")
    bpw = B // NW
    body = functools.partial(_gather_body, bpw, L)
    out = pl.kernel(
        body,
        out_type=jax.ShapeDtypeStruct((B, L, D), jnp.float32),
        mesh=mesh,
        compiler_params=pltpu.CompilerParams(needs_layout_passes=False),
        scratch_types=[
            pltpu.VMEM((L, bpw), jnp.int32),
            pltpu.VMEM((bpw, L), jnp.int32),
            pltpu.VMEM((L, 2 * D), jnp.float32),
            pltpu.VMEM((L, 2 * D), jnp.float32),
            pltpu.VMEM((L, D), jnp.float32),
            pltpu.VMEM((L, D), jnp.float32),
            pltpu.SemaphoreType.DMA,
            pltpu.SemaphoreType.DMA,
            pltpu.SemaphoreType.DMA,
            pltpu.SemaphoreType.DMA,
        ],
    )(table2, wordsT)
    return out
